# baseline (speedup 1.0000x reference)
import functools

import jax
import jax.numpy as jnp
from jax import lax
from jax.experimental import pallas as pl
from jax.experimental.pallas import tpu as pltpu
from jax.experimental.pallas import tpu_sc as plsc

N = 10000
E = 160000
F = 256
HID = 256
NH1 = 8
NEG_SLOPE = 0.2
EPS_BN = 1e-5

NP = 10240
EP = 163840
PAD_NODE = NP - 1

NC = 2
NS = 16
NW = NC * NS
EPW = EP // NW
BLK = 128
NBLK = EPW // BLK
ZROWS = NP // NS
FBLK = 64
FEPW = EP // NS
FNBLK = FEPW // FBLK

RM = 256


def _elu(v):
    return jnp.where(v > 0, v, jnp.exp(jnp.minimum(v, 0.0)) - 1.0)


def _mm_call(x, w, ncols):
    k = x.shape[1]

    def body(x_ref, w_ref, o_ref):
        o_ref[0] = jnp.dot(x_ref[...], w_ref[...],
                           preferred_element_type=jnp.float32)

    return pl.pallas_call(
        body,
        grid=(NP // RM, ncols),
        in_specs=[pl.BlockSpec((RM, k), lambda i, c: (i, 0)),
                  pl.BlockSpec((k, 128), lambda i, c: (0, c))],
        out_specs=pl.BlockSpec((1, RM, 128), lambda i, c: (c, i, 0)),
        out_shape=jax.ShapeDtypeStruct((ncols, NP, 128), jnp.float32),
    )(x, w)


def _alpha_call(h3, amat):
    nch = h3.shape[0]

    def body(amat_ref, h_ref, o_ref):
        hb = h_ref[...]
        hcat = jnp.concatenate([hb[c] for c in range(nch)], axis=1)
        o_ref[...] = jnp.dot(hcat, amat_ref[...],
                             preferred_element_type=jnp.float32)

    return pl.pallas_call(
        body,
        grid=(NP // RM,),
        in_specs=[pl.BlockSpec(amat.shape, lambda i: (0, 0)),
                  pl.BlockSpec((nch, RM, 128), lambda i: (0, i, 0))],
        out_specs=pl.BlockSpec((RM, 32), lambda i: (i, 0)),
        out_shape=jax.ShapeDtypeStruct((NP, 32), jnp.float32),
    )(amat, h3)


def _combine1_call(rs, hp):
    nch = hp.shape[0]

    def body(rs_ref, hp_ref, o_ref):
        r = rs_ref[...]
        rsum = r[0] + r[1]
        h = hp_ref[...]
        cols = []
        for c in range(nch):
            denom = rsum[:, c // 2][:, None] + 1e-16
            cols.append(_elu(h[c] / denom))
        o_ref[...] = jnp.concatenate(cols, axis=1)

    return pl.pallas_call(
        body,
        grid=(NP // RM,),
        in_specs=[pl.BlockSpec((2, RM, 16), lambda i: (0, i, 0)),
                  pl.BlockSpec((nch, RM, 128), lambda i: (0, i, 0))],
        out_specs=pl.BlockSpec((RM, nch * 128), lambda i: (i, 0)),
        out_shape=jax.ShapeDtypeStruct((NP, nch * 128), jnp.float32),
    )(rs, hp)


def _final_call(rs, hp, gamma, beta):
    nch = hp.shape[0]
    inv = float(1.0 / (1.0 + EPS_BN) ** 0.5)

    def body(rs_ref, g_ref, b_ref, hp_ref, o_ref):
        r = rs_ref[...]
        denom = (r[0] + r[1])[:, 0:1] + 1e-16
        h = hp_ref[...]
        s = jnp.concatenate([h[c] for c in range(nch)], axis=1)
        x2 = s / denom
        xn = x2 * (g_ref[...] * inv) + b_ref[...]
        o_ref[...] = _elu(xn)

    return pl.pallas_call(
        body,
        grid=(NP // RM,),
        in_specs=[pl.BlockSpec((2, RM, 16), lambda i: (0, i, 0)),
                  pl.BlockSpec((1, nch * 128), lambda i: (0, 0)),
                  pl.BlockSpec((1, nch * 128), lambda i: (0, 0)),
                  pl.BlockSpec((nch, RM, 128), lambda i: (0, i, 0))],
        out_specs=pl.BlockSpec((RM, nch * 128), lambda i: (i, 0)),
        out_shape=jax.ShapeDtypeStruct((NP, nch * 128), jnp.float32),
    )(rs, gamma.reshape(1, -1), beta.reshape(1, -1), hp)


def _mesh():
    return plsc.VectorSubcoreMesh(core_axis_name="c", subcore_axis_name="s")


def _edgew_call(acat, src2d, dst2d, nh):
    @functools.partial(
        pl.kernel,
        mesh=_mesh(),
        compiler_params=pltpu.CompilerParams(use_tc_tiling_on_sc=False),
        out_type=[jax.ShapeDtypeStruct((EP, 16), jnp.float32),
                  jax.ShapeDtypeStruct((NC, NP, 16), jnp.float32)],
        scratch_types=[
            pltpu.VMEM((NBLK, BLK), jnp.int32),
            pltpu.VMEM((NBLK, BLK), jnp.int32),
            pltpu.VMEM((BLK, 32), jnp.float32),
            pltpu.VMEM((BLK, 32), jnp.float32),
            pltpu.VMEM((BLK, 32), jnp.float32),
            pltpu.VMEM((BLK, 32), jnp.float32),
            pltpu.VMEM((BLK, 16), jnp.float32),
            pltpu.VMEM_SHARED((NP, 16), jnp.float32),
            pltpu.SemaphoreType.DMA,
            pltpu.SemaphoreType.DMA,
            pltpu.SemaphoreType.DMA,
            pltpu.SemaphoreType.DMA,
        ],
    )
    def k(acat_h, src_h, dst_h, wt_out, rs_out,
          s2d, d2d, as0, ad0, as1, ad1, wrow, rsacc, sa0, sb0, sa1, sb1):
        cid = lax.axis_index("c")
        tid = lax.axis_index("s")
        wid = tid * NC + cid

        def zero_wrow(i, carry):
            wrow[i, :] = jnp.zeros((16,), jnp.float32)
            return carry

        lax.fori_loop(0, BLK, zero_wrow, 0)
        for piece in range(ZROWS // BLK):
            pltpu.sync_copy(wrow, rsacc.at[pl.ds(tid * ZROWS + piece * BLK, BLK)])
        plsc.subcore_barrier()

        brow = wid * NBLK
        ebase = wid * EPW
        pltpu.sync_copy(src_h.at[pl.ds(brow, NBLK)], s2d)
        pltpu.sync_copy(dst_h.at[pl.ds(brow, NBLK)], d2d)

        def issue(b, abuf, dbuf, sema, semb):
            pltpu.async_copy(acat_h.at[s2d.at[b]], abuf, sema)
            pltpu.async_copy(acat_h.at[d2d.at[b]], dbuf, semb)

        def wait(b, abuf, dbuf, sema, semb):
            pltpu.make_async_copy(acat_h.at[s2d.at[b]], abuf, sema).wait()
            pltpu.make_async_copy(acat_h.at[d2d.at[b]], dbuf, semb).wait()

        def phase(b, abuf, dbuf, sema, semb, anx, dnx, semna, semnb):
            wait(b, abuf, dbuf, sema, semb)

            @pl.when(b + 1 < NBLK)
            def _():
                issue(b + 1, anx, dnx, semna, semnb)

            def edge(e, c2):
                s = abuf[e, pl.ds(0, 16)]
                d = dbuf[e, pl.ds(16, 16)]
                ev = s + d
                lr = jnp.where(ev > 0, ev, NEG_SLOPE * ev)
                wrow[e, :] = jnp.exp(-lr)
                return c2

            lax.fori_loop(0, BLK, edge, 0)
            pltpu.sync_copy(wrow, rsacc.at[s2d.at[b]], add=True)
            pltpu.sync_copy(wrow, wt_out.at[pl.ds(ebase + b * BLK, BLK)])

        issue(0, as0, ad0, sa0, sb0)

        def pair(g, carry):
            phase(2 * g, as0, ad0, sa0, sb0, as1, ad1, sa1, sb1)
            phase(2 * g + 1, as1, ad1, sa1, sb1, as0, ad0, sa0, sb0)
            return carry

        lax.fori_loop(0, NBLK // 2, pair, 0)
        plsc.subcore_barrier()
        for piece in range(ZROWS // BLK):
            off = tid * ZROWS + piece * BLK
            pltpu.sync_copy(rsacc.at[pl.ds(off, BLK)],
                            rs_out.at[cid, pl.ds(off, BLK)])

    return k(acat, src2d, dst2d)


def _featpass_call(h3, wt, src2d, dst2d, nh):
    ch = h3.shape[0]
    cph = ch // nh
    chc = ch // NC

    @functools.partial(
        pl.kernel,
        mesh=_mesh(),
        compiler_params=pltpu.CompilerParams(use_tc_tiling_on_sc=False),
        out_type=jax.ShapeDtypeStruct((ch, NP, 128), jnp.float32),
        scratch_types=[
            pltpu.VMEM((FNBLK, FBLK), jnp.int32),
            pltpu.VMEM((FNBLK, FBLK), jnp.int32),
            pltpu.VMEM((FBLK, 16), jnp.float32),
            pltpu.VMEM((FBLK, 16), jnp.float32),
            pltpu.VMEM((FBLK, 16), jnp.float32),
            pltpu.VMEM((FBLK, 128), jnp.float32),
            pltpu.VMEM((FBLK, 128), jnp.float32),
            pltpu.VMEM((FBLK, 128), jnp.float32),
            pltpu.VMEM_SHARED((NP, 128), jnp.float32),
        ] + [pltpu.SemaphoreType.DMA] * 9,
    )
    def k(h3_h, wt_h, src_h, dst_h, out_h,
          s2d, d2d, wb0, wb1, wb2, r0, r1, r2, acc,
          sg0, sg1, sg2, sw0, sw1, sw2, ss0, ss1, ss2):
        cid = lax.axis_index("c")
        tid = lax.axis_index("s")
        rbufs = (r0, r1, r2)
        wbufs = (wb0, wb1, wb2)
        sgs = (sg0, sg1, sg2)
        sws = (sw0, sw1, sw2)
        sss = (ss0, ss1, ss2)

        def zero_r0(i, carry):
            for j in range(8):
                r0[i, pl.ds(j * 16, 16)] = jnp.zeros((16,), jnp.float32)
            return carry

        lax.fori_loop(0, FBLK, zero_r0, 0)

        brow = tid * FNBLK
        ebase = tid * FEPW
        pltpu.sync_copy(src_h.at[pl.ds(brow, FNBLK)], s2d)
        pltpu.sync_copy(dst_h.at[pl.ds(brow, FNBLK)], d2d)

        for piece in range(ZROWS // FBLK):
            pltpu.sync_copy(r0, acc.at[pl.ds(tid * ZROWS + piece * FBLK, FBLK)])
        plsc.subcore_barrier()

        def chunk(cl, carry):
            c = cid * chc + cl
            chead = c // cph
            lanes_c = jnp.zeros((16,), jnp.int32) + chead
            hc = h3_h.at[c]

            def issue(b, p):
                pltpu.async_copy(hc.at[d2d.at[b]], rbufs[p], sgs[p])
                pltpu.async_copy(wt_h.at[pl.ds(ebase + b * FBLK, FBLK)],
                                 wbufs[p], sws[p])

            def wait_gather(b, p):
                pltpu.make_async_copy(hc.at[d2d.at[b]], rbufs[p], sgs[p]).wait()
                pltpu.make_async_copy(wt_h.at[pl.ds(ebase + b * FBLK, FBLK)],
                                      wbufs[p], sws[p]).wait()

            def wait_scatter(b, p):
                pltpu.make_async_copy(rbufs[p], acc.at[s2d.at[b]],
                                      sss[p]).wait()

            def phase(b, p, last):
                pn = (p + 1) % 3
                wait_gather(b, p)

                @pl.when(b >= 2)
                def _():
                    wait_scatter(b - 2, pn)

                if not last:
                    @pl.when(b + 1 < FNBLK)
                    def _():
                        issue(b + 1, pn)

                @plsc.parallel_loop(0, FBLK, unroll=4)
                def _(e):
                    wrow = wbufs[p][e, :]
                    wv = wrow[lanes_c]
                    for j in range(8):
                        rbufs[p][e, pl.ds(j * 16, 16)] = (
                            rbufs[p][e, pl.ds(j * 16, 16)] * wv)

                pltpu.async_copy(rbufs[p], acc.at[s2d.at[b]], sss[p],
                                 add=True)

            issue(0, 0)

            def trio(g, c2):
                b = 3 * g
                phase(b, 0, False)
                phase(b + 1, 1, False)
                phase(b + 2, 2, False)
                return c2

            lax.fori_loop(0, (FNBLK - 1) // 3, trio, 0)
            phase(FNBLK - 1, (FNBLK - 1) % 3, True)
            wait_scatter(FNBLK - 2, (FNBLK - 2) % 3)
            wait_scatter(FNBLK - 1, (FNBLK - 1) % 3)
            plsc.subcore_barrier()
            lax.fori_loop(0, FBLK, zero_r0, 0)
            for piece in range(ZROWS // FBLK):
                off = tid * ZROWS + piece * FBLK
                pltpu.sync_copy(acc.at[pl.ds(off, FBLK)],
                                out_h.at[c, pl.ds(off, FBLK)])
                pltpu.sync_copy(r0, acc.at[pl.ds(off, FBLK)])
            plsc.subcore_barrier()
            return carry

        lax.fori_loop(0, chc, chunk, 0)

    return k(h3, wt, src2d, dst2d)


def kernel(x, edge_index, w1, a1, w2, a2, bn_gamma, bn_beta):
    f32 = jnp.float32
    x_pad = jnp.pad(x, ((0, NP - N), (0, 0)))
    pad_idx = jnp.full((EP - E,), PAD_NODE, jnp.int32)
    srcp = jnp.concatenate([edge_index[0], pad_idx])
    dstp = jnp.concatenate([edge_index[1], pad_idx])
    src2d = srcp.reshape(EP // BLK, BLK)
    dst2d = dstp.reshape(EP // BLK, BLK)
    src2f = srcp.reshape(EP // FBLK, FBLK)
    dst2f = dstp.reshape(EP // FBLK, FBLK)

    w_all = jnp.transpose(w1, (1, 0, 2)).reshape(F, NH1 * HID)
    amat1 = jnp.zeros((NH1 * HID, 32), f32)
    for h in range(NH1):
        amat1 = amat1.at[h * HID:(h + 1) * HID, h].set(a1[h, :HID])
        amat1 = amat1.at[h * HID:(h + 1) * HID, 16 + h].set(a1[h, HID:])
    amat2 = (jnp.zeros((HID, 32), f32)
             .at[:, 0].set(a2[:HID])
             .at[:, 16].set(a2[HID:]))

    h1 = _mm_call(x_pad, w_all, ncols=16)
    acat1 = _alpha_call(h1, amat1)
    wt1, rs1 = _edgew_call(acat1, src2d, dst2d, nh=NH1)
    hp1 = _featpass_call(h1, wt1, src2f, dst2f, nh=NH1)
    x1 = _combine1_call(rs1, hp1)

    h2 = _mm_call(x1, w2, ncols=2)
    acat2 = _alpha_call(h2, amat2)
    wt2, rs2 = _edgew_call(acat2, src2d, dst2d, nh=1)
    hp2 = _featpass_call(h2, wt2, src2f, dst2f, nh=1)
    out = _final_call(rs2, hp2, bn_gamma, bn_beta)
    return out[:N]


# speedup vs baseline: 1.0396x; 1.0396x over previous
import functools

import jax
import jax.numpy as jnp
from jax import lax
from jax.experimental import pallas as pl
from jax.experimental.pallas import tpu as pltpu
from jax.experimental.pallas import tpu_sc as plsc

N = 10000
E = 160000
F = 256
HID = 256
NH1 = 8
NEG_SLOPE = 0.2
EPS_BN = 1e-5

NP = 10240
EP = 163840
PAD_NODE = NP - 1

NC = 2
NS = 16
NW = NC * NS
EPW = EP // NW
BLK = 128
NBLK = EPW // BLK
ZROWS = NP // NS
FBLK = 80
FEPW = EP // NS
FNBLK = FEPW // FBLK

RM = 256


def _elu(v):
    return jnp.where(v > 0, v, jnp.exp(jnp.minimum(v, 0.0)) - 1.0)


def _mm_call(x, w, ncols):
    k = x.shape[1]

    def body(x_ref, w_ref, o_ref):
        o_ref[0] = jnp.dot(x_ref[...], w_ref[...],
                           preferred_element_type=jnp.float32)

    return pl.pallas_call(
        body,
        grid=(NP // RM, ncols),
        in_specs=[pl.BlockSpec((RM, k), lambda i, c: (i, 0)),
                  pl.BlockSpec((k, 128), lambda i, c: (0, c))],
        out_specs=pl.BlockSpec((1, RM, 128), lambda i, c: (c, i, 0)),
        out_shape=jax.ShapeDtypeStruct((ncols, NP, 128), jnp.float32),
    )(x, w)


def _alpha_call(h3, amat):
    nch = h3.shape[0]

    def body(amat_ref, h_ref, o_ref):
        hb = h_ref[...]
        hcat = jnp.concatenate([hb[c] for c in range(nch)], axis=1)
        o_ref[...] = jnp.dot(hcat, amat_ref[...],
                             preferred_element_type=jnp.float32)

    return pl.pallas_call(
        body,
        grid=(NP // RM,),
        in_specs=[pl.BlockSpec(amat.shape, lambda i: (0, 0)),
                  pl.BlockSpec((nch, RM, 128), lambda i: (0, i, 0))],
        out_specs=pl.BlockSpec((RM, 32), lambda i: (i, 0)),
        out_shape=jax.ShapeDtypeStruct((NP, 32), jnp.float32),
    )(amat, h3)


def _combine1_call(rs, hp):
    nch = hp.shape[0]

    def body(rs_ref, hp_ref, o_ref):
        r = rs_ref[...]
        rsum = r[0] + r[1]
        h = hp_ref[...]
        cols = []
        for c in range(nch):
            denom = rsum[:, c // 2][:, None] + 1e-16
            cols.append(_elu(h[c] / denom))
        o_ref[...] = jnp.concatenate(cols, axis=1)

    return pl.pallas_call(
        body,
        grid=(NP // RM,),
        in_specs=[pl.BlockSpec((2, RM, 16), lambda i: (0, i, 0)),
                  pl.BlockSpec((nch, RM, 128), lambda i: (0, i, 0))],
        out_specs=pl.BlockSpec((RM, nch * 128), lambda i: (i, 0)),
        out_shape=jax.ShapeDtypeStruct((NP, nch * 128), jnp.float32),
    )(rs, hp)


def _final_call(rs, hp, gamma, beta):
    nch = hp.shape[0]
    inv = float(1.0 / (1.0 + EPS_BN) ** 0.5)

    def body(rs_ref, g_ref, b_ref, hp_ref, o_ref):
        r = rs_ref[...]
        denom = (r[0] + r[1])[:, 0:1] + 1e-16
        h = hp_ref[...]
        s = jnp.concatenate([h[c] for c in range(nch)], axis=1)
        x2 = s / denom
        xn = x2 * (g_ref[...] * inv) + b_ref[...]
        o_ref[...] = _elu(xn)

    return pl.pallas_call(
        body,
        grid=(NP // RM,),
        in_specs=[pl.BlockSpec((2, RM, 16), lambda i: (0, i, 0)),
                  pl.BlockSpec((1, nch * 128), lambda i: (0, 0)),
                  pl.BlockSpec((1, nch * 128), lambda i: (0, 0)),
                  pl.BlockSpec((nch, RM, 128), lambda i: (0, i, 0))],
        out_specs=pl.BlockSpec((RM, nch * 128), lambda i: (i, 0)),
        out_shape=jax.ShapeDtypeStruct((NP, nch * 128), jnp.float32),
    )(rs, gamma.reshape(1, -1), beta.reshape(1, -1), hp)


def _mesh():
    return plsc.VectorSubcoreMesh(core_axis_name="c", subcore_axis_name="s")


def _edgew_call(acat, src2d, dst2d, nh):
    @functools.partial(
        pl.kernel,
        mesh=_mesh(),
        compiler_params=pltpu.CompilerParams(use_tc_tiling_on_sc=False),
        out_type=[jax.ShapeDtypeStruct((EP, 16), jnp.float32),
                  jax.ShapeDtypeStruct((NC, NP, 16), jnp.float32)],
        scratch_types=[
            pltpu.VMEM((NBLK, BLK), jnp.int32),
            pltpu.VMEM((NBLK, BLK), jnp.int32),
            pltpu.VMEM((BLK, 32), jnp.float32),
            pltpu.VMEM((BLK, 32), jnp.float32),
            pltpu.VMEM((BLK, 32), jnp.float32),
            pltpu.VMEM((BLK, 32), jnp.float32),
            pltpu.VMEM((BLK, 16), jnp.float32),
            pltpu.VMEM_SHARED((NP, 16), jnp.float32),
            pltpu.SemaphoreType.DMA,
            pltpu.SemaphoreType.DMA,
            pltpu.SemaphoreType.DMA,
            pltpu.SemaphoreType.DMA,
        ],
    )
    def k(acat_h, src_h, dst_h, wt_out, rs_out,
          s2d, d2d, as0, ad0, as1, ad1, wrow, rsacc, sa0, sb0, sa1, sb1):
        cid = lax.axis_index("c")
        tid = lax.axis_index("s")
        wid = tid * NC + cid

        def zero_wrow(i, carry):
            wrow[i, :] = jnp.zeros((16,), jnp.float32)
            return carry

        lax.fori_loop(0, BLK, zero_wrow, 0)
        for piece in range(ZROWS // BLK):
            pltpu.sync_copy(wrow, rsacc.at[pl.ds(tid * ZROWS + piece * BLK, BLK)])
        plsc.subcore_barrier()

        brow = wid * NBLK
        ebase = wid * EPW
        pltpu.sync_copy(src_h.at[pl.ds(brow, NBLK)], s2d)
        pltpu.sync_copy(dst_h.at[pl.ds(brow, NBLK)], d2d)

        def issue(b, abuf, dbuf, sema, semb):
            pltpu.async_copy(acat_h.at[s2d.at[b]], abuf, sema)
            pltpu.async_copy(acat_h.at[d2d.at[b]], dbuf, semb)

        def wait(b, abuf, dbuf, sema, semb):
            pltpu.make_async_copy(acat_h.at[s2d.at[b]], abuf, sema).wait()
            pltpu.make_async_copy(acat_h.at[d2d.at[b]], dbuf, semb).wait()

        def phase(b, abuf, dbuf, sema, semb, anx, dnx, semna, semnb):
            wait(b, abuf, dbuf, sema, semb)

            @pl.when(b + 1 < NBLK)
            def _():
                issue(b + 1, anx, dnx, semna, semnb)

            def edge(e, c2):
                s = abuf[e, pl.ds(0, 16)]
                d = dbuf[e, pl.ds(16, 16)]
                ev = s + d
                lr = jnp.where(ev > 0, ev, NEG_SLOPE * ev)
                wrow[e, :] = jnp.exp(-lr)
                return c2

            lax.fori_loop(0, BLK, edge, 0)
            pltpu.sync_copy(wrow, rsacc.at[s2d.at[b]], add=True)
            pltpu.sync_copy(wrow, wt_out.at[pl.ds(ebase + b * BLK, BLK)])

        issue(0, as0, ad0, sa0, sb0)

        def pair(g, carry):
            phase(2 * g, as0, ad0, sa0, sb0, as1, ad1, sa1, sb1)
            phase(2 * g + 1, as1, ad1, sa1, sb1, as0, ad0, sa0, sb0)
            return carry

        lax.fori_loop(0, NBLK // 2, pair, 0)
        plsc.subcore_barrier()
        for piece in range(ZROWS // BLK):
            off = tid * ZROWS + piece * BLK
            pltpu.sync_copy(rsacc.at[pl.ds(off, BLK)],
                            rs_out.at[cid, pl.ds(off, BLK)])

    return k(acat, src2d, dst2d)


def _featpass_call(h3, wt, src2d, dst2d, nh):
    ch = h3.shape[0]
    cph = ch // nh
    chc = ch // NC
    RB = FBLK * 128 * 4
    WB = FBLK * 16 * 4

    @functools.partial(
        pl.kernel,
        mesh=_mesh(),
        compiler_params=pltpu.CompilerParams(use_tc_tiling_on_sc=False),
        out_type=jax.ShapeDtypeStruct((ch, NP, 128), jnp.float32),
        scratch_types=[
            pltpu.VMEM((FNBLK, FBLK), jnp.int32),
            pltpu.VMEM((FNBLK, FBLK), jnp.int32),
            pltpu.VMEM((FBLK, 16), jnp.float32),
            pltpu.VMEM((FBLK, 16), jnp.float32),
            pltpu.VMEM((FBLK, 128), jnp.float32),
            pltpu.VMEM((FBLK, 128), jnp.float32),
            pltpu.VMEM_SHARED((NP, 128), jnp.float32),
        ] + [pltpu.SemaphoreType.DMA] * 6,
    )
    def k(h3_h, wt_h, src_h, dst_h, out_h,
          s2d, d2d, wb0, wb1, r0, r1, acc, sg0, sg1, sw0, sw1, ss0, ss1):
        cid = lax.axis_index("c")
        tid = lax.axis_index("s")

        def zero_r0(i, carry):
            for j in range(8):
                r0[i, pl.ds(j * 16, 16)] = jnp.zeros((16,), jnp.float32)
            return carry

        lax.fori_loop(0, FBLK, zero_r0, 0)

        brow = tid * FNBLK
        ebase = tid * FEPW
        pltpu.sync_copy(src_h.at[pl.ds(brow, FNBLK)], s2d)
        pltpu.sync_copy(dst_h.at[pl.ds(brow, FNBLK)], d2d)

        for piece in range(ZROWS // FBLK):
            pltpu.sync_copy(r0, acc.at[pl.ds(tid * ZROWS + piece * FBLK, FBLK)])
        plsc.subcore_barrier()

        def chunk(cl, carry):
            c = cid * chc + cl
            chead = c // cph
            lanes_c = jnp.zeros((16,), jnp.int32) + chead
            hc = h3_h.at[c]

            def issue(b, rbuf, wbuf, semg, semw):
                pltpu.async_copy(hc.at[d2d.at[b]], rbuf, semg)
                pltpu.async_copy(wt_h.at[pl.ds(ebase + b * FBLK, FBLK)],
                                 wbuf, semw)

            def phase(b, rbuf, wbuf, semg, semw, sems,
                      rnx, wnx, semgn, semwn, semsn):
                pltpu.make_async_copy(hc.at[d2d.at[b]], rbuf, semg).wait()
                pltpu.make_async_copy(wt_h.at[pl.ds(ebase + b * FBLK, FBLK)],
                                      wbuf, semw).wait()

                @pl.when(b >= 1)
                def _():
                    pltpu.make_async_copy(
                        rnx, acc.at[s2d.at[b - 1]], semsn).wait()

                @pl.when(b + 1 < FNBLK)
                def _():
                    issue(b + 1, rnx, wnx, semgn, semwn)

                @plsc.parallel_loop(0, FBLK, unroll=8)
                def _(e):
                    wrow = wbuf[e, :]
                    wv = wrow[lanes_c]
                    for j in range(8):
                        rbuf[e, pl.ds(j * 16, 16)] = (
                            rbuf[e, pl.ds(j * 16, 16)] * wv)

                pltpu.async_copy(rbuf, acc.at[s2d.at[b]], sems, add=True)

            issue(0, r0, wb0, sg0, sw0)

            def pair(g, c2):
                phase(2 * g, r0, wb0, sg0, sw0, ss0, r1, wb1, sg1, sw1, ss1)
                phase(2 * g + 1, r1, wb1, sg1, sw1, ss1, r0, wb0, sg0, sw0, ss0)
                return c2

            lax.fori_loop(0, FNBLK // 2, pair, 0)
            pltpu.make_async_copy(r1, acc.at[s2d.at[FNBLK - 1]], ss1).wait()
            plsc.subcore_barrier()
            lax.fori_loop(0, FBLK, zero_r0, 0)
            for piece in range(ZROWS // FBLK):
                off = tid * ZROWS + piece * FBLK
                pltpu.sync_copy(acc.at[pl.ds(off, FBLK)],
                                out_h.at[c, pl.ds(off, FBLK)])
                pltpu.sync_copy(r0, acc.at[pl.ds(off, FBLK)])
            plsc.subcore_barrier()
            return carry

        lax.fori_loop(0, chc, chunk, 0)

    return k(h3, wt, src2d, dst2d)


def kernel(x, edge_index, w1, a1, w2, a2, bn_gamma, bn_beta):
    f32 = jnp.float32
    x_pad = jnp.pad(x, ((0, NP - N), (0, 0)))
    pad_idx = jnp.full((EP - E,), PAD_NODE, jnp.int32)
    srcp = jnp.concatenate([edge_index[0], pad_idx])
    dstp = jnp.concatenate([edge_index[1], pad_idx])
    src2d = srcp.reshape(EP // BLK, BLK)
    dst2d = dstp.reshape(EP // BLK, BLK)
    src2f = srcp.reshape(EP // FBLK, FBLK)
    dst2f = dstp.reshape(EP // FBLK, FBLK)

    w_all = jnp.transpose(w1, (1, 0, 2)).reshape(F, NH1 * HID)
    amat1 = jnp.zeros((NH1 * HID, 32), f32)
    for h in range(NH1):
        amat1 = amat1.at[h * HID:(h + 1) * HID, h].set(a1[h, :HID])
        amat1 = amat1.at[h * HID:(h + 1) * HID, 16 + h].set(a1[h, HID:])
    amat2 = (jnp.zeros((HID, 32), f32)
             .at[:, 0].set(a2[:HID])
             .at[:, 16].set(a2[HID:]))

    h1 = _mm_call(x_pad, w_all, ncols=16)
    acat1 = _alpha_call(h1, amat1)
    wt1, rs1 = _edgew_call(acat1, src2d, dst2d, nh=NH1)
    hp1 = _featpass_call(h1, wt1, src2f, dst2f, nh=NH1)
    x1 = _combine1_call(rs1, hp1)

    h2 = _mm_call(x1, w2, ncols=2)
    acat2 = _alpha_call(h2, amat2)
    wt2, rs2 = _edgew_call(acat2, src2d, dst2d, nh=1)
    hp2 = _featpass_call(h2, wt2, src2f, dst2f, nh=1)
    out = _final_call(rs2, hp2, bn_gamma, bn_beta)
    return out[:N]
